# Initial kernel scaffold; baseline (speedup 1.0000x reference)
import jax
import jax.numpy as jnp
from jax.experimental import pallas as pl


def kernel(x, edge_index, batch, W0, b0, W1, b1, W2, b2, W3, b3, lin_W, lin_b):
    raise NotImplementedError("write your pallas kernel here")


# speedup vs baseline: 14.2721x; 14.2721x over previous
import functools

import jax
import jax.numpy as jnp
from jax import lax
from jax.experimental import pallas as pl
from jax.experimental.pallas import tpu as pltpu
from jax.experimental.pallas import tpu_sc as plsc

N = 10000
NP = 10240
F = 128
H = 30
HP = 32
G = 64
E = 320000
NC, NS = 2, 16
NW = NC * NS
ET = 10240
EP = ET * NW
FG = 8
FW = HP // FG
NR = 4
ER = EP // NR
CE = 2048
NCH = ER // CE

_f32 = jnp.float32


def _deg_body(dst_hbm, out_hbm, dst_v, acc_v):
    cid = lax.axis_index("c")
    sid = lax.axis_index("s")
    wid = cid * NS + sid
    zeros16 = jnp.zeros((16,), _f32)
    ones16 = jnp.ones((16,), _f32)

    def zfill(i, carry):
        acc_v[pl.ds(i * 16, 16)] = zeros16
        return carry
    lax.fori_loop(0, NP // 16, zfill, 0)

    pltpu.sync_copy(dst_hbm.at[pl.ds(wid * ET, ET)], dst_v)

    def body(i, carry):
        idx = dst_v[pl.ds(i * 16, 16)]
        plsc.addupdate_scatter(acc_v, [idx], ones16)
        return carry
    lax.fori_loop(0, ET // 16, body, 0)

    pltpu.sync_copy(acc_v, out_hbm.at[wid])


def _agg_body(hs_hbm, src_hbm, dst_hbm, zero_hbm, out_hbm,
              hs_t, acc_t, srcc_v, dstc_v):
    cid = lax.axis_index("c")
    sid = lax.axis_index("s")
    g = sid % FG
    r = cid * (NS // FG) + sid // FG

    pltpu.sync_copy(hs_hbm.at[g], hs_t)
    pltpu.sync_copy(zero_hbm, acc_t)

    def chunk(c, carry):
        eb = r * ER + c * CE
        pltpu.sync_copy(src_hbm.at[pl.ds(eb, CE)], srcc_v)
        pltpu.sync_copy(dst_hbm.at[pl.ds(eb, CE)], dstc_v)

        def inner(i, carry2):
            s16 = srcc_v[pl.ds(i * 16, 16)]
            d16 = dstc_v[pl.ds(i * 16, 16)]
            for f in range(FW):
                v = plsc.load_gather(hs_t, [s16 + f * NP])
                plsc.addupdate_scatter(acc_t, [d16 + f * NP], v)
            return carry2
        lax.fori_loop(0, CE // 16, inner, 0)
        return carry
    lax.fori_loop(0, NCH, chunk, 0)

    pltpu.sync_copy(acc_t, out_hbm.at[r, g])


@functools.cache
def _sc_kernels():
    mesh = plsc.VectorSubcoreMesh(
        core_axis_name="c", subcore_axis_name="s",
        num_cores=NC, num_subcores=NS)
    cp = pltpu.CompilerParams(needs_layout_passes=False)
    deg = pl.kernel(
        _deg_body,
        out_type=jax.ShapeDtypeStruct((NW, NP), _f32),
        mesh=mesh,
        compiler_params=cp,
        scratch_types=[
            pltpu.VMEM((ET,), jnp.int32),
            pltpu.VMEM((NP,), _f32),
        ],
    )
    agg = pl.kernel(
        _agg_body,
        out_type=jax.ShapeDtypeStruct((NR, FG, NP * FW), _f32),
        mesh=mesh,
        compiler_params=cp,
        scratch_types=[
            pltpu.VMEM((NP * FW,), _f32),
            pltpu.VMEM((NP * FW,), _f32),
            pltpu.VMEM((CE,), jnp.int32),
            pltpu.VMEM((CE,), jnp.int32),
        ],
    )
    return deg, agg


def _pooled(hT, bt):
    grow = lax.broadcasted_iota(jnp.int32, (G, 1), 0)

    def g_body(g, acc):
        mx = jnp.max(jnp.where(bt == g, hT, -jnp.inf), axis=1)
        return jnp.where(grow == g, mx[None], acc)
    return lax.fori_loop(0, G, g_body, jnp.full((G, HP), -jnp.inf, _f32))


def _assemble(p_ref, hs_ref):
    rows = []
    for g in range(FG):
        rows.append(p_ref[0, g] + p_ref[1, g] + p_ref[2, g] + p_ref[3, g]
                    + hs_ref[g])
    return jnp.concatenate(rows, axis=0)


def _tcb_body(xpt_ref, w0t_ref, degp_ref, hs0_ref, dinv_ref):
    deg = jnp.sum(degp_ref[...], axis=0) + 1.0
    dinv = lax.rsqrt(jnp.maximum(deg, 1.0))[None]
    dinv_ref[...] = dinv
    hT = jnp.dot(w0t_ref[...], xpt_ref[...], preferred_element_type=_f32)
    hs0_ref[...] = (dinv * hT).reshape(FG, FW, NP)


_tcb = pl.pallas_call(
    _tcb_body,
    out_shape=(jax.ShapeDtypeStruct((FG, FW, NP), _f32),
               jax.ShapeDtypeStruct((1, NP), _f32)))


def _tcc_body(p_ref, hsin_ref, dinv_ref, b_ref, wnt_ref, batch_ref,
              hsout_ref, pooled_ref):
    preT = _assemble(p_ref, hsin_ref)
    hT = jnp.maximum(dinv_ref[...] * preT + b_ref[...], 0.0)
    cols = lax.broadcasted_iota(jnp.int32, (1, NP), 1)
    hT = jnp.where(cols < N, hT, 0.0)
    pooled_ref[...] = _pooled(hT, batch_ref[...])
    hsout_ref[...] = (dinv_ref[...] * jnp.dot(
        wnt_ref[...], hT, preferred_element_type=_f32)).reshape(FG, FW, NP)


_tcc = pl.pallas_call(
    _tcc_body,
    out_shape=(jax.ShapeDtypeStruct((FG, FW, NP), _f32),
               jax.ShapeDtypeStruct((G, HP), _f32)))


def _tcd_body(p_ref, hsin_ref, dinv_ref, b_ref, batch_ref, pooled_ref,
              lw_ref, lb_ref, out_ref):
    preT = _assemble(p_ref, hsin_ref)
    hT = dinv_ref[...] * preT + b_ref[...]
    p3 = _pooled(hT, batch_ref[...])
    z = (jnp.dot(pooled_ref[0], lw_ref[0], preferred_element_type=_f32)
         + jnp.dot(pooled_ref[1], lw_ref[1], preferred_element_type=_f32)
         + jnp.dot(pooled_ref[2], lw_ref[2], preferred_element_type=_f32)
         + jnp.dot(p3, lw_ref[3], preferred_element_type=_f32))
    out_ref[...] = z + lb_ref[...]


_tcd = pl.pallas_call(
    _tcd_body,
    out_shape=jax.ShapeDtypeStruct((G, 2), _f32))


def kernel(x, edge_index, batch, W0, b0, W1, b1, W2, b2, W3, b3, lin_W, lin_b):
    src = edge_index[0].astype(jnp.int32)
    dst = edge_index[1].astype(jnp.int32)
    pad_e = EP - E
    srcp = jnp.concatenate([src, jnp.full((pad_e,), N, jnp.int32)])
    dstp = jnp.concatenate([dst, jnp.full((pad_e,), N, jnp.int32)])

    xpt = jnp.zeros((F, NP), _f32).at[:, :N].set(x.astype(_f32).T)
    batchp = jnp.concatenate(
        [batch.astype(jnp.int32), jnp.full((NP - N,), G, jnp.int32)]
    ).reshape(1, NP)
    zero_nf = jnp.zeros((FW * NP,), _f32)

    def padwt(w, rr, cc):
        return jnp.zeros((rr, cc), _f32).at[:w.shape[0], :w.shape[1]].set(
            w.astype(_f32)).T

    W0t = padwt(W0, F, HP)
    W1t, W2t, W3t = (padwt(w, HP, HP) for w in (W1, W2, W3))
    b0p, b1p, b2p, b3p = (
        jnp.zeros((HP, 1), _f32).at[:H, 0].set(b.astype(_f32))
        for b in (b0, b1, b2, b3))
    lwp = jnp.zeros((4, HP, 2), _f32)
    for k in range(4):
        lwp = lwp.at[k, :H, :].set(lin_W[k * H:(k + 1) * H].astype(_f32))
    lbp = lin_b.astype(_f32).reshape(1, 2)

    _deg_kernel, _agg_kernel = _sc_kernels()

    def agg(hs):
        p = _agg_kernel(hs.reshape(FG, FW * NP), srcp, dstp, zero_nf)
        return p.reshape(NR, FG, FW, NP)

    degp = _deg_kernel(dstp)
    hs0, dinv = _tcb(xpt, W0t, degp)
    p0 = agg(hs0)
    hs1, pooled0 = _tcc(p0, hs0, dinv, b0p, W1t, batchp)
    p1 = agg(hs1)
    hs2, pooled1 = _tcc(p1, hs1, dinv, b1p, W2t, batchp)
    p2 = agg(hs2)
    hs3, pooled2 = _tcc(p2, hs2, dinv, b2p, W3t, batchp)
    p3 = agg(hs3)
    pooled012 = jnp.stack([pooled0, pooled1, pooled2])
    return _tcd(p3, hs3, dinv, b3p, batchp, pooled012, lwp, lbp)


# speedup vs baseline: 17.5167x; 1.2273x over previous
import functools

import jax
import jax.numpy as jnp
from jax import lax
from jax.experimental import pallas as pl
from jax.experimental.pallas import tpu as pltpu
from jax.experimental.pallas import tpu_sc as plsc

N = 10000
NP = 10240
F = 128
H = 30
HP = 32
G = 64
E = 320000
NC, NS = 2, 16
NW = NC * NS
ET = 10240
EP = ET * NW
FG = 8
FW = HP // FG
NR = 4
ER = EP // NR
CE = 8192
NCH = ER // CE

_f32 = jnp.float32


def _deg_body(dst_hbm, out_hbm, dst_v, acc_v):
    cid = lax.axis_index("c")
    sid = lax.axis_index("s")
    wid = cid * NS + sid
    zeros16 = jnp.zeros((16,), _f32)
    ones16 = jnp.ones((16,), _f32)

    def zfill(i, carry):
        acc_v[pl.ds(i * 16, 16)] = zeros16
        return carry
    lax.fori_loop(0, NP // 16, zfill, 0)

    pltpu.sync_copy(dst_hbm.at[pl.ds(wid * ET, ET)], dst_v)

    def body(i, carry):
        idx = dst_v[pl.ds(i * 16, 16)]
        plsc.addupdate_scatter(acc_v, [idx], ones16)
        return carry
    lax.fori_loop(0, ET // 16, body, 0)

    pltpu.sync_copy(acc_v, out_hbm.at[wid])


def _agg_body(hs_hbm, src_hbm, dst_hbm, zero_hbm, out_hbm,
              hs_t, acc_t, srcc0, dstc0, srcc1, dstc1, sem0, sem1):
    cid = lax.axis_index("c")
    sid = lax.axis_index("s")
    g = sid % FG
    r = cid * (NS // FG) + sid // FG

    bufs = ((srcc0, dstc0, sem0), (srcc1, dstc1, sem1))

    def start(ci, b):
        sv, dv, sem = bufs[b]
        eb = r * ER + ci * CE
        pltpu.async_copy(src_hbm.at[pl.ds(eb, CE)], sv, sem)
        pltpu.async_copy(dst_hbm.at[pl.ds(eb, CE)], dv, sem)

    def drain(b):
        sv, dv, sem = bufs[b]
        pltpu.make_async_copy(src_hbm.at[pl.ds(0, CE)], sv, sem).wait()
        pltpu.make_async_copy(dst_hbm.at[pl.ds(0, CE)], dv, sem).wait()

    start(0, 0)
    pltpu.sync_copy(hs_hbm.at[g], hs_t)
    pltpu.sync_copy(zero_hbm, acc_t)

    def outer(c2, carry):
        for b in range(2):
            ci = c2 * 2 + b
            drain(b)

            @pl.when(ci + 1 < NCH)
            def _():
                start(ci + 1, 1 - b)

            sv, dv, _ = bufs[b]

            def inner(i, carry2):
                s16 = sv[pl.ds(i * 16, 16)]
                d16 = dv[pl.ds(i * 16, 16)]
                for f in range(FW):
                    v = plsc.load_gather(hs_t, [s16 + f * NP])
                    plsc.addupdate_scatter(acc_t, [d16 + f * NP], v)
                return carry2
            lax.fori_loop(0, CE // 16, inner, 0)
        return carry
    lax.fori_loop(0, NCH // 2, outer, 0)

    pltpu.sync_copy(acc_t, out_hbm.at[r, g])


@functools.cache
def _sc_kernels():
    mesh = plsc.VectorSubcoreMesh(
        core_axis_name="c", subcore_axis_name="s",
        num_cores=NC, num_subcores=NS)
    cp = pltpu.CompilerParams(needs_layout_passes=False)
    deg = pl.kernel(
        _deg_body,
        out_type=jax.ShapeDtypeStruct((NW, NP), _f32),
        mesh=mesh,
        compiler_params=cp,
        scratch_types=[
            pltpu.VMEM((ET,), jnp.int32),
            pltpu.VMEM((NP,), _f32),
        ],
    )
    agg = pl.kernel(
        _agg_body,
        out_type=jax.ShapeDtypeStruct((NR, FG, NP * FW), _f32),
        mesh=mesh,
        compiler_params=cp,
        scratch_types=[
            pltpu.VMEM((NP * FW,), _f32),
            pltpu.VMEM((NP * FW,), _f32),
            pltpu.VMEM((CE,), jnp.int32),
            pltpu.VMEM((CE,), jnp.int32),
            pltpu.VMEM((CE,), jnp.int32),
            pltpu.VMEM((CE,), jnp.int32),
            pltpu.SemaphoreType.DMA,
            pltpu.SemaphoreType.DMA,
        ],
    )
    return deg, agg


def _pooled(hT, bt):
    grow = lax.broadcasted_iota(jnp.int32, (G, 1), 0)

    def g_body(g, acc):
        mx = jnp.max(jnp.where(bt == g, hT, -jnp.inf), axis=1)
        return jnp.where(grow == g, mx[None], acc)
    return lax.fori_loop(0, G, g_body, jnp.full((G, HP), -jnp.inf, _f32))


def _assemble(p_ref, hs_ref):
    rows = []
    for g in range(FG):
        rows.append(p_ref[0, g] + p_ref[1, g] + p_ref[2, g] + p_ref[3, g]
                    + hs_ref[g])
    return jnp.concatenate(rows, axis=0)


def _tcb_body(xpt_ref, w0t_ref, degp_ref, hs0_ref, dinv_ref):
    deg = jnp.sum(degp_ref[...], axis=0) + 1.0
    dinv = lax.rsqrt(jnp.maximum(deg, 1.0))[None]
    dinv_ref[...] = dinv
    hT = jnp.dot(w0t_ref[...], xpt_ref[...], preferred_element_type=_f32)
    hs0_ref[...] = (dinv * hT).reshape(FG, FW, NP)


_tcb = pl.pallas_call(
    _tcb_body,
    out_shape=(jax.ShapeDtypeStruct((FG, FW, NP), _f32),
               jax.ShapeDtypeStruct((1, NP), _f32)))


def _tcc_body(p_ref, hsin_ref, dinv_ref, b_ref, wnt_ref, batch_ref,
              hsout_ref, pooled_ref):
    preT = _assemble(p_ref, hsin_ref)
    hT = jnp.maximum(dinv_ref[...] * preT + b_ref[...], 0.0)
    cols = lax.broadcasted_iota(jnp.int32, (1, NP), 1)
    hT = jnp.where(cols < N, hT, 0.0)
    pooled_ref[...] = _pooled(hT, batch_ref[...])
    hsout_ref[...] = (dinv_ref[...] * jnp.dot(
        wnt_ref[...], hT, preferred_element_type=_f32)).reshape(FG, FW, NP)


_tcc = pl.pallas_call(
    _tcc_body,
    out_shape=(jax.ShapeDtypeStruct((FG, FW, NP), _f32),
               jax.ShapeDtypeStruct((G, HP), _f32)))


def _tcd_body(p_ref, hsin_ref, dinv_ref, b_ref, batch_ref, pooled_ref,
              lw_ref, lb_ref, out_ref):
    preT = _assemble(p_ref, hsin_ref)
    hT = dinv_ref[...] * preT + b_ref[...]
    p3 = _pooled(hT, batch_ref[...])
    z = (jnp.dot(pooled_ref[0], lw_ref[0], preferred_element_type=_f32)
         + jnp.dot(pooled_ref[1], lw_ref[1], preferred_element_type=_f32)
         + jnp.dot(pooled_ref[2], lw_ref[2], preferred_element_type=_f32)
         + jnp.dot(p3, lw_ref[3], preferred_element_type=_f32))
    out_ref[...] = z + lb_ref[...]


_tcd = pl.pallas_call(
    _tcd_body,
    out_shape=jax.ShapeDtypeStruct((G, 2), _f32))


def kernel(x, edge_index, batch, W0, b0, W1, b1, W2, b2, W3, b3, lin_W, lin_b):
    src = edge_index[0].astype(jnp.int32)
    dst = edge_index[1].astype(jnp.int32)
    pad_e = EP - E
    srcp = jnp.concatenate([src, jnp.full((pad_e,), N, jnp.int32)])
    dstp = jnp.concatenate([dst, jnp.full((pad_e,), N, jnp.int32)])

    xpt = jnp.zeros((F, NP), _f32).at[:, :N].set(x.astype(_f32).T)
    batchp = jnp.concatenate(
        [batch.astype(jnp.int32), jnp.full((NP - N,), G, jnp.int32)]
    ).reshape(1, NP)
    zero_nf = jnp.zeros((FW * NP,), _f32)

    def padwt(w, rr, cc):
        return jnp.zeros((rr, cc), _f32).at[:w.shape[0], :w.shape[1]].set(
            w.astype(_f32)).T

    W0t = padwt(W0, F, HP)
    W1t, W2t, W3t = (padwt(w, HP, HP) for w in (W1, W2, W3))
    b0p, b1p, b2p, b3p = (
        jnp.zeros((HP, 1), _f32).at[:H, 0].set(b.astype(_f32))
        for b in (b0, b1, b2, b3))
    lwp = jnp.zeros((4, HP, 2), _f32)
    for k in range(4):
        lwp = lwp.at[k, :H, :].set(lin_W[k * H:(k + 1) * H].astype(_f32))
    lbp = lin_b.astype(_f32).reshape(1, 2)

    _deg_kernel, _agg_kernel = _sc_kernels()

    def agg(hs):
        p = _agg_kernel(hs.reshape(FG, FW * NP), srcp, dstp, zero_nf)
        return p.reshape(NR, FG, FW, NP)

    degp = _deg_kernel(dstp)
    hs0, dinv = _tcb(xpt, W0t, degp)
    p0 = agg(hs0)
    hs1, pooled0 = _tcc(p0, hs0, dinv, b0p, W1t, batchp)
    p1 = agg(hs1)
    hs2, pooled1 = _tcc(p1, hs1, dinv, b1p, W2t, batchp)
    p2 = agg(hs2)
    hs3, pooled2 = _tcc(p2, hs2, dinv, b2p, W3t, batchp)
    p3 = agg(hs3)
    pooled012 = jnp.stack([pooled0, pooled1, pooled2])
    return _tcd(p3, hs3, dinv, b3p, batchp, pooled012, lwp, lbp)


# speedup vs baseline: 25.4498x; 1.4529x over previous
import functools

import jax
import jax.numpy as jnp
from jax import lax
from jax.experimental import pallas as pl
from jax.experimental.pallas import tpu as pltpu
from jax.experimental.pallas import tpu_sc as plsc

N = 10000
NP = 10240
F = 128
H = 30
HP = 32
G = 64
E = 320000
NC, NS = 2, 16
NW = NC * NS
ET = 10240
EP = ET * NW
FG = 8
FW = HP // FG
NR = 4
ER = EP // NR
CE = 8192
NCH = ER // CE

_f32 = jnp.float32


def _deg_body(dst_hbm, out_hbm, dst_v, acc_v):
    cid = lax.axis_index("c")
    sid = lax.axis_index("s")
    wid = cid * NS + sid
    zeros16 = jnp.zeros((16,), _f32)
    ones16 = jnp.ones((16,), _f32)

    def zfill(i, carry):
        acc_v[pl.ds(i * 16, 16)] = zeros16
        return carry
    lax.fori_loop(0, NP // 16, zfill, 0)

    pltpu.sync_copy(dst_hbm.at[pl.ds(wid * ET, ET)], dst_v)

    def body(i, carry):
        idx = dst_v[pl.ds(i * 16, 16)]
        plsc.addupdate_scatter(acc_v, [idx], ones16)
        return carry
    lax.fori_loop(0, ET // 16, body, 0)

    pltpu.sync_copy(acc_v, out_hbm.at[wid])


def _agg_body(hs_hbm, src_hbm, dst_hbm, zero_hbm, out_hbm,
              hs_t, acc_t, srcc0, dstc0, srcc1, dstc1, sem0, sem1):
    cid = lax.axis_index("c")
    sid = lax.axis_index("s")
    g = sid % FG
    r = cid * (NS // FG) + sid // FG

    bufs = ((srcc0, dstc0, sem0), (srcc1, dstc1, sem1))

    def start(ci, b):
        sv, dv, sem = bufs[b]
        eb = r * ER + ci * CE
        pltpu.async_copy(src_hbm.at[pl.ds(eb, CE)], sv, sem)
        pltpu.async_copy(dst_hbm.at[pl.ds(eb, CE)], dv, sem)

    def drain(b):
        sv, dv, sem = bufs[b]
        pltpu.make_async_copy(src_hbm.at[pl.ds(0, CE)], sv, sem).wait()
        pltpu.make_async_copy(dst_hbm.at[pl.ds(0, CE)], dv, sem).wait()

    start(0, 0)
    pltpu.sync_copy(hs_hbm.at[g], hs_t)
    pltpu.sync_copy(zero_hbm, acc_t)

    def outer(c2, carry):
        for b in range(2):
            ci = c2 * 2 + b
            drain(b)

            @pl.when(ci + 1 < NCH)
            def _():
                start(ci + 1, 1 - b)

            sv, dv, _ = bufs[b]

            def inner(i, carry2):
                idxs = []
                for u in range(4):
                    base = (i * 4 + u) * 16
                    idxs.append((sv[pl.ds(base, 16)], dv[pl.ds(base, 16)]))
                for s16, d16 in idxs:
                    vals = [plsc.load_gather(hs_t, [s16 + f * NP])
                            for f in range(FW)]
                    for f, v in enumerate(vals):
                        plsc.addupdate_scatter(acc_t, [d16 + f * NP], v)
                return carry2
            lax.fori_loop(0, CE // 64, inner, 0)
        return carry
    lax.fori_loop(0, NCH // 2, outer, 0)

    pltpu.sync_copy(acc_t, out_hbm.at[r, g])


@functools.cache
def _sc_kernels():
    mesh = plsc.VectorSubcoreMesh(
        core_axis_name="c", subcore_axis_name="s",
        num_cores=NC, num_subcores=NS)
    cp = pltpu.CompilerParams(needs_layout_passes=False)
    deg = pl.kernel(
        _deg_body,
        out_type=jax.ShapeDtypeStruct((NW, NP), _f32),
        mesh=mesh,
        compiler_params=cp,
        scratch_types=[
            pltpu.VMEM((ET,), jnp.int32),
            pltpu.VMEM((NP,), _f32),
        ],
    )
    agg = pl.kernel(
        _agg_body,
        out_type=jax.ShapeDtypeStruct((NR, FG, NP * FW), _f32),
        mesh=mesh,
        compiler_params=cp,
        scratch_types=[
            pltpu.VMEM((NP * FW,), _f32),
            pltpu.VMEM((NP * FW,), _f32),
            pltpu.VMEM((CE,), jnp.int32),
            pltpu.VMEM((CE,), jnp.int32),
            pltpu.VMEM((CE,), jnp.int32),
            pltpu.VMEM((CE,), jnp.int32),
            pltpu.SemaphoreType.DMA,
            pltpu.SemaphoreType.DMA,
        ],
    )
    return deg, agg


def _pooled(hT, bt):
    grow = lax.broadcasted_iota(jnp.int32, (G, 1), 0)

    def g_body(g, acc):
        mx = jnp.max(jnp.where(bt == g, hT, -jnp.inf), axis=1)
        return jnp.where(grow == g, mx[None], acc)
    return lax.fori_loop(0, G, g_body, jnp.full((G, HP), -jnp.inf, _f32))


def _assemble(p_ref, hs_ref):
    rows = []
    for g in range(FG):
        rows.append(p_ref[0, g] + p_ref[1, g] + p_ref[2, g] + p_ref[3, g]
                    + hs_ref[g])
    return jnp.concatenate(rows, axis=0)


def _tcb_body(xpt_ref, w0t_ref, degp_ref, hs0_ref, dinv_ref):
    deg = jnp.sum(degp_ref[...], axis=0) + 1.0
    dinv = lax.rsqrt(jnp.maximum(deg, 1.0))[None]
    dinv_ref[...] = dinv
    hT = jnp.dot(w0t_ref[...], xpt_ref[...], preferred_element_type=_f32)
    hs0_ref[...] = (dinv * hT).reshape(FG, FW, NP)


_tcb = pl.pallas_call(
    _tcb_body,
    out_shape=(jax.ShapeDtypeStruct((FG, FW, NP), _f32),
               jax.ShapeDtypeStruct((1, NP), _f32)))


def _tcc_body(p_ref, hsin_ref, dinv_ref, b_ref, wnt_ref, batch_ref,
              hsout_ref, pooled_ref):
    preT = _assemble(p_ref, hsin_ref)
    hT = jnp.maximum(dinv_ref[...] * preT + b_ref[...], 0.0)
    cols = lax.broadcasted_iota(jnp.int32, (1, NP), 1)
    hT = jnp.where(cols < N, hT, 0.0)
    pooled_ref[...] = _pooled(hT, batch_ref[...])
    hsout_ref[...] = (dinv_ref[...] * jnp.dot(
        wnt_ref[...], hT, preferred_element_type=_f32)).reshape(FG, FW, NP)


_tcc = pl.pallas_call(
    _tcc_body,
    out_shape=(jax.ShapeDtypeStruct((FG, FW, NP), _f32),
               jax.ShapeDtypeStruct((G, HP), _f32)))


def _tcd_body(p_ref, hsin_ref, dinv_ref, b_ref, batch_ref, pooled_ref,
              lw_ref, lb_ref, out_ref):
    preT = _assemble(p_ref, hsin_ref)
    hT = dinv_ref[...] * preT + b_ref[...]
    p3 = _pooled(hT, batch_ref[...])
    z = (jnp.dot(pooled_ref[0], lw_ref[0], preferred_element_type=_f32)
         + jnp.dot(pooled_ref[1], lw_ref[1], preferred_element_type=_f32)
         + jnp.dot(pooled_ref[2], lw_ref[2], preferred_element_type=_f32)
         + jnp.dot(p3, lw_ref[3], preferred_element_type=_f32))
    out_ref[...] = z + lb_ref[...]


_tcd = pl.pallas_call(
    _tcd_body,
    out_shape=jax.ShapeDtypeStruct((G, 2), _f32))


def kernel(x, edge_index, batch, W0, b0, W1, b1, W2, b2, W3, b3, lin_W, lin_b):
    src = edge_index[0].astype(jnp.int32)
    dst = edge_index[1].astype(jnp.int32)
    pad_e = EP - E
    srcp = jnp.concatenate([src, jnp.full((pad_e,), N, jnp.int32)])
    dstp = jnp.concatenate([dst, jnp.full((pad_e,), N, jnp.int32)])

    xpt = jnp.zeros((F, NP), _f32).at[:, :N].set(x.astype(_f32).T)
    batchp = jnp.concatenate(
        [batch.astype(jnp.int32), jnp.full((NP - N,), G, jnp.int32)]
    ).reshape(1, NP)
    zero_nf = jnp.zeros((FW * NP,), _f32)

    def padwt(w, rr, cc):
        return jnp.zeros((rr, cc), _f32).at[:w.shape[0], :w.shape[1]].set(
            w.astype(_f32)).T

    W0t = padwt(W0, F, HP)
    W1t, W2t, W3t = (padwt(w, HP, HP) for w in (W1, W2, W3))
    b0p, b1p, b2p, b3p = (
        jnp.zeros((HP, 1), _f32).at[:H, 0].set(b.astype(_f32))
        for b in (b0, b1, b2, b3))
    lwp = jnp.zeros((4, HP, 2), _f32)
    for k in range(4):
        lwp = lwp.at[k, :H, :].set(lin_W[k * H:(k + 1) * H].astype(_f32))
    lbp = lin_b.astype(_f32).reshape(1, 2)

    _deg_kernel, _agg_kernel = _sc_kernels()

    def agg(hs):
        p = _agg_kernel(hs.reshape(FG, FW * NP), srcp, dstp, zero_nf)
        return p.reshape(NR, FG, FW, NP)

    degp = _deg_kernel(dstp)
    hs0, dinv = _tcb(xpt, W0t, degp)
    p0 = agg(hs0)
    hs1, pooled0 = _tcc(p0, hs0, dinv, b0p, W1t, batchp)
    p1 = agg(hs1)
    hs2, pooled1 = _tcc(p1, hs1, dinv, b1p, W2t, batchp)
    p2 = agg(hs2)
    hs3, pooled2 = _tcc(p2, hs2, dinv, b2p, W3t, batchp)
    p3 = agg(hs3)
    pooled012 = jnp.stack([pooled0, pooled1, pooled2])
    return _tcd(p3, hs3, dinv, b3p, batchp, pooled012, lwp, lbp)


# speedup vs baseline: 25.7290x; 1.0110x over previous
import functools

import jax
import jax.numpy as jnp
from jax import lax
from jax.experimental import pallas as pl
from jax.experimental.pallas import tpu as pltpu
from jax.experimental.pallas import tpu_sc as plsc

N = 10000
NP = 10240
F = 128
H = 30
HP = 32
G = 64
E = 320000
NC, NS = 2, 16
NW = NC * NS
ET = 10240
EP = ET * NW
FG = 8
FW = HP // FG
NR = 4
ER = EP // NR
CE = 8192
NCH = ER // CE

_f32 = jnp.float32


def _deg_body(dst_hbm, out_hbm, dst_v, acc_v):
    cid = lax.axis_index("c")
    sid = lax.axis_index("s")
    wid = cid * NS + sid
    zeros16 = jnp.zeros((16,), _f32)
    ones16 = jnp.ones((16,), _f32)

    def zfill(i, carry):
        acc_v[pl.ds(i * 16, 16)] = zeros16
        return carry
    lax.fori_loop(0, NP // 16, zfill, 0)

    pltpu.sync_copy(dst_hbm.at[pl.ds(wid * ET, ET)], dst_v)

    def body(i, carry):
        idx = dst_v[pl.ds(i * 16, 16)]
        plsc.addupdate_scatter(acc_v, [idx], ones16)
        return carry
    lax.fori_loop(0, ET // 16, body, 0)

    pltpu.sync_copy(acc_v, out_hbm.at[wid])


def _agg_body(hs_hbm, src_hbm, dst_hbm, zero_hbm, out_hbm,
              hs_t, acc_t, srcc0, dstc0, srcc1, dstc1, sem0, sem1):
    cid = lax.axis_index("c")
    sid = lax.axis_index("s")
    g = sid % FG
    r = cid * (NS // FG) + sid // FG

    bufs = ((srcc0, dstc0, sem0), (srcc1, dstc1, sem1))

    def start(ci, b):
        sv, dv, sem = bufs[b]
        eb = r * ER + ci * CE
        pltpu.async_copy(src_hbm.at[pl.ds(eb, CE)], sv, sem)
        pltpu.async_copy(dst_hbm.at[pl.ds(eb, CE)], dv, sem)

    def drain(b):
        sv, dv, sem = bufs[b]
        pltpu.make_async_copy(src_hbm.at[pl.ds(0, CE)], sv, sem).wait()
        pltpu.make_async_copy(dst_hbm.at[pl.ds(0, CE)], dv, sem).wait()

    start(0, 0)
    pltpu.sync_copy(hs_hbm.at[g], hs_t)
    pltpu.sync_copy(zero_hbm, acc_t)

    def outer(c2, carry):
        for b in range(2):
            ci = c2 * 2 + b
            drain(b)

            @pl.when(ci + 1 < NCH)
            def _():
                start(ci + 1, 1 - b)

            sv, dv, _ = bufs[b]

            def inner(i, carry2):
                idxs = []
                for u in range(8):
                    base = (i * 8 + u) * 16
                    idxs.append((sv[pl.ds(base, 16)], dv[pl.ds(base, 16)]))
                for s16, d16 in idxs:
                    vals = [plsc.load_gather(hs_t, [s16 + f * NP])
                            for f in range(FW)]
                    for f, v in enumerate(vals):
                        plsc.addupdate_scatter(acc_t, [d16 + f * NP], v)
                return carry2
            lax.fori_loop(0, CE // 128, inner, 0)
        return carry
    lax.fori_loop(0, NCH // 2, outer, 0)

    pltpu.sync_copy(acc_t, out_hbm.at[r, g])


@functools.cache
def _sc_kernels():
    mesh = plsc.VectorSubcoreMesh(
        core_axis_name="c", subcore_axis_name="s",
        num_cores=NC, num_subcores=NS)
    cp = pltpu.CompilerParams(needs_layout_passes=False)
    deg = pl.kernel(
        _deg_body,
        out_type=jax.ShapeDtypeStruct((NW, NP), _f32),
        mesh=mesh,
        compiler_params=cp,
        scratch_types=[
            pltpu.VMEM((ET,), jnp.int32),
            pltpu.VMEM((NP,), _f32),
        ],
    )
    agg = pl.kernel(
        _agg_body,
        out_type=jax.ShapeDtypeStruct((NR, FG, NP * FW), _f32),
        mesh=mesh,
        compiler_params=cp,
        scratch_types=[
            pltpu.VMEM((NP * FW,), _f32),
            pltpu.VMEM((NP * FW,), _f32),
            pltpu.VMEM((CE,), jnp.int32),
            pltpu.VMEM((CE,), jnp.int32),
            pltpu.VMEM((CE,), jnp.int32),
            pltpu.VMEM((CE,), jnp.int32),
            pltpu.SemaphoreType.DMA,
            pltpu.SemaphoreType.DMA,
        ],
    )
    return deg, agg


def _pooled(hT, bt):
    grow = lax.broadcasted_iota(jnp.int32, (G, 1), 0)

    def g_body(g, acc):
        mx = jnp.max(jnp.where(bt == g, hT, -jnp.inf), axis=1)
        return jnp.where(grow == g, mx[None], acc)
    return lax.fori_loop(0, G, g_body, jnp.full((G, HP), -jnp.inf, _f32))


def _assemble(p_ref, hs_ref):
    rows = []
    for g in range(FG):
        rows.append(p_ref[0, g] + p_ref[1, g] + p_ref[2, g] + p_ref[3, g]
                    + hs_ref[g])
    return jnp.concatenate(rows, axis=0)


def _tcb_body(xpt_ref, w0t_ref, degp_ref, hs0_ref, dinv_ref):
    deg = jnp.sum(degp_ref[...], axis=0) + 1.0
    dinv = lax.rsqrt(jnp.maximum(deg, 1.0))[None]
    dinv_ref[...] = dinv
    hT = jnp.dot(w0t_ref[...], xpt_ref[...], preferred_element_type=_f32)
    hs0_ref[...] = (dinv * hT).reshape(FG, FW, NP)


_tcb = pl.pallas_call(
    _tcb_body,
    out_shape=(jax.ShapeDtypeStruct((FG, FW, NP), _f32),
               jax.ShapeDtypeStruct((1, NP), _f32)))


def _tcc_body(p_ref, hsin_ref, dinv_ref, b_ref, wnt_ref, batch_ref,
              hsout_ref, pooled_ref):
    preT = _assemble(p_ref, hsin_ref)
    hT = jnp.maximum(dinv_ref[...] * preT + b_ref[...], 0.0)
    cols = lax.broadcasted_iota(jnp.int32, (1, NP), 1)
    hT = jnp.where(cols < N, hT, 0.0)
    pooled_ref[...] = _pooled(hT, batch_ref[...])
    hsout_ref[...] = (dinv_ref[...] * jnp.dot(
        wnt_ref[...], hT, preferred_element_type=_f32)).reshape(FG, FW, NP)


_tcc = pl.pallas_call(
    _tcc_body,
    out_shape=(jax.ShapeDtypeStruct((FG, FW, NP), _f32),
               jax.ShapeDtypeStruct((G, HP), _f32)))


def _tcd_body(p_ref, hsin_ref, dinv_ref, b_ref, batch_ref, pooled_ref,
              lw_ref, lb_ref, out_ref):
    preT = _assemble(p_ref, hsin_ref)
    hT = dinv_ref[...] * preT + b_ref[...]
    p3 = _pooled(hT, batch_ref[...])
    z = (jnp.dot(pooled_ref[0], lw_ref[0], preferred_element_type=_f32)
         + jnp.dot(pooled_ref[1], lw_ref[1], preferred_element_type=_f32)
         + jnp.dot(pooled_ref[2], lw_ref[2], preferred_element_type=_f32)
         + jnp.dot(p3, lw_ref[3], preferred_element_type=_f32))
    out_ref[...] = z + lb_ref[...]


_tcd = pl.pallas_call(
    _tcd_body,
    out_shape=jax.ShapeDtypeStruct((G, 2), _f32))


def kernel(x, edge_index, batch, W0, b0, W1, b1, W2, b2, W3, b3, lin_W, lin_b):
    src = edge_index[0].astype(jnp.int32)
    dst = edge_index[1].astype(jnp.int32)
    pad_e = EP - E
    srcp = jnp.concatenate([src, jnp.full((pad_e,), N, jnp.int32)])
    dstp = jnp.concatenate([dst, jnp.full((pad_e,), N, jnp.int32)])

    xpt = jnp.zeros((F, NP), _f32).at[:, :N].set(x.astype(_f32).T)
    batchp = jnp.concatenate(
        [batch.astype(jnp.int32), jnp.full((NP - N,), G, jnp.int32)]
    ).reshape(1, NP)
    zero_nf = jnp.zeros((FW * NP,), _f32)

    def padwt(w, rr, cc):
        return jnp.zeros((rr, cc), _f32).at[:w.shape[0], :w.shape[1]].set(
            w.astype(_f32)).T

    W0t = padwt(W0, F, HP)
    W1t, W2t, W3t = (padwt(w, HP, HP) for w in (W1, W2, W3))
    b0p, b1p, b2p, b3p = (
        jnp.zeros((HP, 1), _f32).at[:H, 0].set(b.astype(_f32))
        for b in (b0, b1, b2, b3))
    lwp = jnp.zeros((4, HP, 2), _f32)
    for k in range(4):
        lwp = lwp.at[k, :H, :].set(lin_W[k * H:(k + 1) * H].astype(_f32))
    lbp = lin_b.astype(_f32).reshape(1, 2)

    _deg_kernel, _agg_kernel = _sc_kernels()

    def agg(hs):
        p = _agg_kernel(hs.reshape(FG, FW * NP), srcp, dstp, zero_nf)
        return p.reshape(NR, FG, FW, NP)

    degp = _deg_kernel(dstp)
    hs0, dinv = _tcb(xpt, W0t, degp)
    p0 = agg(hs0)
    hs1, pooled0 = _tcc(p0, hs0, dinv, b0p, W1t, batchp)
    p1 = agg(hs1)
    hs2, pooled1 = _tcc(p1, hs1, dinv, b1p, W2t, batchp)
    p2 = agg(hs2)
    hs3, pooled2 = _tcc(p2, hs2, dinv, b2p, W3t, batchp)
    p3 = agg(hs3)
    pooled012 = jnp.stack([pooled0, pooled1, pooled2])
    return _tcd(p3, hs3, dinv, b3p, batchp, pooled012, lwp, lbp)
